# baseline (speedup 1.0000x reference)
import jax
import jax.numpy as jnp
from jax import lax
from jax.experimental import pallas as pl
from jax.experimental.pallas import tpu as pltpu
from jax.experimental.pallas import tpu_sc as plsc

_COLS = 22
_VOCAB = 12
_D = 128
_B = 4096
_REG = 0.01
_NPAIR = _COLS * _COLS

_UPAD = 16
_NROWS = _COLS * _UPAD
_NCOLS = _COLS * _VOCAB
_TABLE_WORDS = _NROWS * _NCOLS

_NC = 2
_NS = 16
_LANES = 16
_NW = _NC * _NS
_BPW = _B // _NW
_GROUPS = _BPW // _LANES
_NACC = 4


def _table_kernel(wp_ref, bp_ref, embp_ref, ep_ref, fc_ref, t_ref, sq_ref):
    wp = wp_ref[0, 0]
    bp = bp_ref[0, 0]
    s_r = lax.broadcasted_iota(jnp.int32, (_NCOLS, _NROWS), 0)
    s_c = lax.broadcasted_iota(jnp.int32, (_NCOLS, _NROWS), 1)
    sel = (s_c == (s_r // _VOCAB) * _UPAD + s_r % _VOCAB).astype(
        jnp.float32)
    e = lax.dot_general(sel, ep_ref[...], (((1,), (0,)), ((), ())),
                        preferred_element_type=jnp.float32)
    tt = e * wp + bp
    sq_ref[...] = jnp.sum(e * e, axis=1, keepdims=True)

    w = fc_ref[...]
    c = jnp.maximum(jnp.sqrt(jnp.sum(w * w, axis=1, keepdims=True)), 1.0)
    wn = w / c

    r_ids = lax.broadcasted_iota(jnp.int32, (_NPAIR, _NPAIR), 0)
    c_ids = lax.broadcasted_iota(jnp.int32, (_NPAIR, _NPAIR), 1)
    pmat = (c_ids == (r_ids % _COLS) * _COLS + r_ids // _COLS).astype(
        jnp.float32)
    wt = lax.dot_general(pmat, wn, (((1,), (0,)), ((), ())),
                         preferred_element_type=jnp.float32)
    d_ids = lax.broadcasted_iota(jnp.int32, (_NPAIR, 1), 0)
    diag = (d_ids % _COLS) == (d_ids // _COLS)
    wsym = wn + jnp.where(diag, 0.0, wt)

    rr_ids = lax.broadcasted_iota(jnp.int32, (_NCOLS, _COLS), 0) // _VOCAB
    rc_ids = lax.broadcasted_iota(jnp.int32, (_NCOLS, _COLS), 1)
    rmat = (rr_ids == rc_ids).astype(jnp.float32)

    reps = [
        lax.dot_general(
            rmat, wsym[i * _COLS:(i + 1) * _COLS, :],
            (((1,), (0,)), ((), ())), preferred_element_type=jnp.float32)
        for i in range(_COLS)
    ]
    cms = [tt * reps[i] for i in range(_COLS)]
    ttis = [embp_ref[i] * wp + bp for i in range(_COLS)]
    blks = [
        lax.dot_general(
            ttis[i], cms[i], (((1,), (1,)), ((), ())),
            preferred_element_type=jnp.float32)
        for i in range(_COLS)
    ]
    for i in range(_COLS):
        t_ref[pl.ds(i * _UPAD, _UPAD), :] = blks[i]


_CHUNKS = (0, 1, 3, 5, 8, 12, 17, 22)


def _sc_kernel(table_hbm, feat_hbm, sq_hbm, out_hbm, part_hbm,
               table_v, feat_v, sq_v, out_v, acc_v, accs_v, *sems):
    wid = lax.axis_index("s") * _NC + lax.axis_index("c")
    base = wid * _BPW

    cps = []
    for k in range(len(_CHUNKS) - 1):
        w0 = _CHUNKS[k] * _UPAD * _NCOLS
        w1 = _CHUNKS[k + 1] * _UPAD * _NCOLS
        cps.append(pltpu.async_copy(
            table_hbm.at[pl.ds(w0, w1 - w0)],
            table_v.at[pl.ds(w0, w1 - w0)], sems[k]))
    pltpu.sync_copy(feat_hbm.at[:, pl.ds(base, _BPW)], feat_v)
    pltpu.sync_copy(sq_hbm, sq_v)
    zeros = jnp.zeros((_LANES,), jnp.float32)
    for i in range(_COLS):
        acc_v[pl.ds(i * _LANES, _LANES)] = zeros

    for k in range(len(_CHUNKS) - 1):
        cps[k].wait()
        i0, i1 = _CHUNKS[k], _CHUNKS[k + 1]

        def group_body(g, carry, i0=i0, i1=i1, first=(k == 0)):
            g16 = g * _LANES
            fj2 = {j: feat_v[j, pl.ds(g16, _LANES)] + (_VOCAB * j)
                   for j in range(i0, _COLS)}
            accs = [jnp.zeros((_LANES,), jnp.float32)
                    for _ in range(_NACC)]
            n = 0
            if first:
                for i in range(_COLS):
                    fi = (feat_v[i, pl.ds(g16, _LANES)] + (_VOCAB * i)
                          if i not in fj2 else fj2[i])
                    sqg = plsc.load_gather(sq_v, [fi])
                    plsc.addupdate(acc_v.at[pl.ds(i * _LANES, _LANES)],
                                   sqg)
            for i in range(i0, i1):
                hi = fj2[i] * _NCOLS + ((_UPAD - _VOCAB) * _NCOLS * i)
                for j in range(i, _COLS):
                    idx = hi + fj2[j]
                    accs[n % _NACC] = accs[n % _NACC] + plsc.load_gather(
                        table_v, [idx])
                    n += 1
            tot = (accs[0] + accs[1]) + (accs[2] + accs[3])
            if first:
                out_v[pl.ds(g16, _LANES)] = tot
            else:
                plsc.addupdate(out_v.at[pl.ds(g16, _LANES)], tot)
            return carry

        lax.fori_loop(0, _GROUPS, group_body, 0)

    pltpu.sync_copy(out_v, out_hbm.at[pl.ds(base, _BPW)])
    lane = lax.iota(jnp.int32, _LANES)
    lo = jnp.zeros((_LANES,), jnp.float32)
    hi_v = jnp.zeros((_LANES,), jnp.float32)
    for i in range(_COLS):
        s_i = jnp.sum(acc_v[pl.ds(i * _LANES, _LANES)])
        if i < _LANES:
            lo = jnp.where(lane == i, s_i, lo)
        else:
            hi_v = jnp.where(lane == (i - _LANES), s_i, hi_v)
    accs_v[pl.ds(0, _LANES)] = lo
    accs_v[pl.ds(_LANES, _LANES)] = hi_v
    pltpu.sync_copy(accs_v, part_hbm.at[wid])


def _regs_kernel(p_ref, out_ref):
    x = p_ref[...]
    s = jnp.sum(x, axis=0, keepdims=True)
    lane = lax.broadcasted_iota(jnp.int32, (1, 32), 1)
    per_col = jnp.where(lane < _COLS, s, 0.0)
    out_ref[0, 0] = jnp.sum(jnp.sqrt(per_col)) * jnp.float32(
        2 * _COLS * _REG)


def kernel(features, emb, fc_w, w_p, b_p):
    embp = jnp.pad(emb, ((0, 0), (0, _UPAD - _VOCAB), (0, 0)))
    ep2 = embp.reshape(_NROWS, _D)
    wp2 = jnp.reshape(w_p, (1, 1))
    bp2 = jnp.reshape(b_p, (1, 1))

    table, sq = pl.pallas_call(
        _table_kernel,
        in_specs=[
            pl.BlockSpec(memory_space=pltpu.SMEM),
            pl.BlockSpec(memory_space=pltpu.SMEM),
            pl.BlockSpec(memory_space=pltpu.VMEM),
            pl.BlockSpec(memory_space=pltpu.VMEM),
            pl.BlockSpec(memory_space=pltpu.VMEM),
        ],
        out_shape=[
            jax.ShapeDtypeStruct((_NROWS, _NCOLS), jnp.float32),
            jax.ShapeDtypeStruct((_NCOLS, 1), jnp.float32),
        ],
    )(wp2, bp2, embp, ep2, fc_w)

    mesh = plsc.VectorSubcoreMesh(
        core_axis_name="c", subcore_axis_name="s",
        num_cores=_NC, num_subcores=_NS)
    sc_fn = pl.kernel(
        _sc_kernel,
        out_type=[
            jax.ShapeDtypeStruct((_B,), jnp.float32),
            jax.ShapeDtypeStruct((_NW, 32), jnp.float32),
        ],
        mesh=mesh,
        compiler_params=pltpu.CompilerParams(
            needs_layout_passes=False, skip_device_barrier=True),
        scratch_types=[
            pltpu.VMEM((_TABLE_WORDS,), jnp.float32),
            pltpu.VMEM((_COLS, _BPW), jnp.int32),
            pltpu.VMEM((_NCOLS,), jnp.float32),
            pltpu.VMEM((_BPW,), jnp.float32),
            pltpu.VMEM((_NROWS,), jnp.float32),
            pltpu.VMEM((32,), jnp.float32),
        ] + [pltpu.SemaphoreType.DMA] * (len(_CHUNKS) - 1),
    )
    out_flat, part = sc_fn(table.reshape(_TABLE_WORDS), features,
                           sq.reshape(_NCOLS))

    regs2 = pl.pallas_call(
        _regs_kernel,
        out_shape=jax.ShapeDtypeStruct((1, 1), jnp.float32),
        out_specs=pl.BlockSpec(memory_space=pltpu.SMEM),
    )(part)

    return out_flat.reshape(_B, 1), jnp.reshape(regs2, ())


# speedup vs baseline: 1.0051x; 1.0051x over previous
import jax
import jax.numpy as jnp
from jax import lax
from jax.experimental import pallas as pl
from jax.experimental.pallas import tpu as pltpu
from jax.experimental.pallas import tpu_sc as plsc

_COLS = 22
_VOCAB = 12
_D = 128
_B = 4096
_REG = 0.01
_NPAIR = _COLS * _COLS

_UPAD = 16
_NROWS = _COLS * _UPAD
_NCOLS = _COLS * _VOCAB
_TABLE_WORDS = _NROWS * _NCOLS

_NC = 2
_NS = 16
_LANES = 16
_NW = _NC * _NS
_BPW = _B // _NW
_GROUPS = _BPW // _LANES
_NACC = 4


def _table_kernel(wp_ref, bp_ref, embp_ref, ep_ref, fc_ref, t_ref, sq_ref):
    wp = wp_ref[0, 0]
    bp = bp_ref[0, 0]
    s_r = lax.broadcasted_iota(jnp.int32, (_NCOLS, _NROWS), 0)
    s_c = lax.broadcasted_iota(jnp.int32, (_NCOLS, _NROWS), 1)
    sel = (s_c == (s_r // _VOCAB) * _UPAD + s_r % _VOCAB).astype(
        jnp.float32)
    e = lax.dot_general(sel, ep_ref[...], (((1,), (0,)), ((), ())),
                        preferred_element_type=jnp.float32)
    tt = e * wp + bp
    sq_ref[...] = jnp.sum(e * e, axis=1, keepdims=True)

    w = fc_ref[...]
    c = jnp.maximum(jnp.sqrt(jnp.sum(w * w, axis=1, keepdims=True)), 1.0)
    wn = w / c

    r_ids = lax.broadcasted_iota(jnp.int32, (_NPAIR, _NPAIR), 0)
    c_ids = lax.broadcasted_iota(jnp.int32, (_NPAIR, _NPAIR), 1)
    pmat = (c_ids == (r_ids % _COLS) * _COLS + r_ids // _COLS).astype(
        jnp.float32)
    wt = lax.dot_general(pmat, wn, (((1,), (0,)), ((), ())),
                         preferred_element_type=jnp.float32)
    d_ids = lax.broadcasted_iota(jnp.int32, (_NPAIR, 1), 0)
    diag = (d_ids % _COLS) == (d_ids // _COLS)
    wsym = wn + jnp.where(diag, 0.0, wt)

    rr_ids = lax.broadcasted_iota(jnp.int32, (_NCOLS, _COLS), 0) // _VOCAB
    rc_ids = lax.broadcasted_iota(jnp.int32, (_NCOLS, _COLS), 1)
    rmat = (rr_ids == rc_ids).astype(jnp.float32)

    reps = [
        lax.dot_general(
            rmat, wsym[i * _COLS:(i + 1) * _COLS, :],
            (((1,), (0,)), ((), ())), preferred_element_type=jnp.float32)
        for i in range(_COLS)
    ]
    cms = [tt * reps[i] for i in range(_COLS)]
    ttis = [embp_ref[i] * wp + bp for i in range(_COLS)]
    blks = [
        lax.dot_general(
            ttis[i], cms[i], (((1,), (1,)), ((), ())),
            preferred_element_type=jnp.float32)
        for i in range(_COLS)
    ]
    for i in range(_COLS):
        t_ref[pl.ds(i * _UPAD, _UPAD), :] = blks[i]


_CHUNKS = (0, 2, 5, 9, 14, 22)


def _sc_kernel(table_hbm, feat_hbm, sq_hbm, out_hbm, part_hbm,
               table_v, feat_v, sq_v, out_v, acc_v, accs_v, *sems):
    wid = lax.axis_index("s") * _NC + lax.axis_index("c")
    base = wid * _BPW

    cps = []
    for k in range(len(_CHUNKS) - 1):
        w0 = _CHUNKS[k] * _UPAD * _NCOLS
        w1 = _CHUNKS[k + 1] * _UPAD * _NCOLS
        cps.append(pltpu.async_copy(
            table_hbm.at[pl.ds(w0, w1 - w0)],
            table_v.at[pl.ds(w0, w1 - w0)], sems[k]))
    pltpu.sync_copy(feat_hbm.at[:, pl.ds(base, _BPW)], feat_v)
    pltpu.sync_copy(sq_hbm, sq_v)
    zeros = jnp.zeros((_LANES,), jnp.float32)
    for i in range(_COLS):
        acc_v[pl.ds(i * _LANES, _LANES)] = zeros

    for k in range(len(_CHUNKS) - 1):
        cps[k].wait()
        i0, i1 = _CHUNKS[k], _CHUNKS[k + 1]

        def group_body(g, carry, i0=i0, i1=i1, first=(k == 0)):
            g16 = g * _LANES
            fj2 = {j: feat_v[j, pl.ds(g16, _LANES)] + (_VOCAB * j)
                   for j in range(i0, _COLS)}
            accs = [jnp.zeros((_LANES,), jnp.float32)
                    for _ in range(_NACC)]
            n = 0
            if first:
                for i in range(_COLS):
                    fi = (feat_v[i, pl.ds(g16, _LANES)] + (_VOCAB * i)
                          if i not in fj2 else fj2[i])
                    sqg = plsc.load_gather(sq_v, [fi])
                    plsc.addupdate(acc_v.at[pl.ds(i * _LANES, _LANES)],
                                   sqg)
            for i in range(i0, i1):
                hi = fj2[i] * _NCOLS + ((_UPAD - _VOCAB) * _NCOLS * i)
                for j in range(i, _COLS):
                    idx = hi + fj2[j]
                    accs[n % _NACC] = accs[n % _NACC] + plsc.load_gather(
                        table_v, [idx])
                    n += 1
            tot = (accs[0] + accs[1]) + (accs[2] + accs[3])
            if first:
                out_v[pl.ds(g16, _LANES)] = tot
            else:
                plsc.addupdate(out_v.at[pl.ds(g16, _LANES)], tot)
            return carry

        lax.fori_loop(0, _GROUPS, group_body, 0)

    pltpu.sync_copy(out_v, out_hbm.at[pl.ds(base, _BPW)])
    lane = lax.iota(jnp.int32, _LANES)
    lo = jnp.zeros((_LANES,), jnp.float32)
    hi_v = jnp.zeros((_LANES,), jnp.float32)
    for i in range(_COLS):
        s_i = jnp.sum(acc_v[pl.ds(i * _LANES, _LANES)])
        if i < _LANES:
            lo = jnp.where(lane == i, s_i, lo)
        else:
            hi_v = jnp.where(lane == (i - _LANES), s_i, hi_v)
    accs_v[pl.ds(0, _LANES)] = lo
    accs_v[pl.ds(_LANES, _LANES)] = hi_v
    pltpu.sync_copy(accs_v, part_hbm.at[wid])


def _regs_kernel(p_ref, out_ref):
    x = p_ref[...]
    s = jnp.sum(x, axis=0, keepdims=True)
    lane = lax.broadcasted_iota(jnp.int32, (1, 32), 1)
    per_col = jnp.where(lane < _COLS, s, 0.0)
    out_ref[0, 0] = jnp.sum(jnp.sqrt(per_col)) * jnp.float32(
        2 * _COLS * _REG)


def kernel(features, emb, fc_w, w_p, b_p):
    embp = jnp.pad(emb, ((0, 0), (0, _UPAD - _VOCAB), (0, 0)))
    ep2 = embp.reshape(_NROWS, _D)
    wp2 = jnp.reshape(w_p, (1, 1))
    bp2 = jnp.reshape(b_p, (1, 1))

    table, sq = pl.pallas_call(
        _table_kernel,
        in_specs=[
            pl.BlockSpec(memory_space=pltpu.SMEM),
            pl.BlockSpec(memory_space=pltpu.SMEM),
            pl.BlockSpec(memory_space=pltpu.VMEM),
            pl.BlockSpec(memory_space=pltpu.VMEM),
            pl.BlockSpec(memory_space=pltpu.VMEM),
        ],
        out_shape=[
            jax.ShapeDtypeStruct((_NROWS, _NCOLS), jnp.float32),
            jax.ShapeDtypeStruct((_NCOLS, 1), jnp.float32),
        ],
    )(wp2, bp2, embp, ep2, fc_w)

    mesh = plsc.VectorSubcoreMesh(
        core_axis_name="c", subcore_axis_name="s",
        num_cores=_NC, num_subcores=_NS)
    sc_fn = pl.kernel(
        _sc_kernel,
        out_type=[
            jax.ShapeDtypeStruct((_B,), jnp.float32),
            jax.ShapeDtypeStruct((_NW, 32), jnp.float32),
        ],
        mesh=mesh,
        compiler_params=pltpu.CompilerParams(
            needs_layout_passes=False, skip_device_barrier=True),
        scratch_types=[
            pltpu.VMEM((_TABLE_WORDS,), jnp.float32),
            pltpu.VMEM((_COLS, _BPW), jnp.int32),
            pltpu.VMEM((_NCOLS,), jnp.float32),
            pltpu.VMEM((_BPW,), jnp.float32),
            pltpu.VMEM((_NROWS,), jnp.float32),
            pltpu.VMEM((32,), jnp.float32),
        ] + [pltpu.SemaphoreType.DMA] * (len(_CHUNKS) - 1),
    )
    out_flat, part = sc_fn(table.reshape(_TABLE_WORDS), features,
                           sq.reshape(_NCOLS))

    regs2 = pl.pallas_call(
        _regs_kernel,
        out_shape=jax.ShapeDtypeStruct((1, 1), jnp.float32),
        out_specs=pl.BlockSpec(memory_space=pltpu.SMEM),
    )(part)

    return out_flat.reshape(_B, 1), jnp.reshape(regs2, ())


# speedup vs baseline: 1.0320x; 1.0268x over previous
import jax
import jax.numpy as jnp
from jax import lax
from jax.experimental import pallas as pl
from jax.experimental.pallas import tpu as pltpu
from jax.experimental.pallas import tpu_sc as plsc

_COLS = 22
_VOCAB = 12
_D = 128
_B = 4096
_REG = 0.01
_NPAIR = _COLS * _COLS

_UPAD = 16
_NROWS = _COLS * _UPAD
_NCOLS = _COLS * _VOCAB
_TABLE_WORDS = _NROWS * _NCOLS

_NC = 2
_NS = 16
_LANES = 16
_NW = _NC * _NS
_BPW = _B // _NW
_GROUPS = _BPW // _LANES
_NACC = 4


def _table_kernel(wp_ref, bp_ref, embp_ref, ep_ref, fc_ref, t_ref, sq_ref):
    wp = wp_ref[0, 0]
    bp = bp_ref[0, 0]
    s_r = lax.broadcasted_iota(jnp.int32, (_NCOLS, _NROWS), 0)
    s_c = lax.broadcasted_iota(jnp.int32, (_NCOLS, _NROWS), 1)
    sel = (s_c == (s_r // _VOCAB) * _UPAD + s_r % _VOCAB).astype(
        jnp.float32)
    e = lax.dot_general(sel, ep_ref[...], (((1,), (0,)), ((), ())),
                        preferred_element_type=jnp.float32)
    tt = e * wp + bp
    sq_ref[...] = jnp.sum(e * e, axis=1, keepdims=True)

    w = fc_ref[...]
    c = jnp.maximum(jnp.sqrt(jnp.sum(w * w, axis=1, keepdims=True)), 1.0)
    wn = w / c

    r_ids = lax.broadcasted_iota(jnp.int32, (_NPAIR, _NPAIR), 0)
    c_ids = lax.broadcasted_iota(jnp.int32, (_NPAIR, _NPAIR), 1)
    pmat = (c_ids == (r_ids % _COLS) * _COLS + r_ids // _COLS).astype(
        jnp.float32)
    wt = lax.dot_general(pmat, wn, (((1,), (0,)), ((), ())),
                         preferred_element_type=jnp.float32)
    d_ids = lax.broadcasted_iota(jnp.int32, (_NPAIR, 1), 0)
    diag = (d_ids % _COLS) == (d_ids // _COLS)
    wsym = wn + jnp.where(diag, 0.0, wt)

    rr_ids = lax.broadcasted_iota(jnp.int32, (_NCOLS, _COLS), 0) // _VOCAB
    rc_ids = lax.broadcasted_iota(jnp.int32, (_NCOLS, _COLS), 1)
    rmat = (rr_ids == rc_ids).astype(jnp.float32)

    reps = [
        lax.dot_general(
            rmat, wsym[i * _COLS:(i + 1) * _COLS, :],
            (((1,), (0,)), ((), ())), preferred_element_type=jnp.float32)
        for i in range(_COLS)
    ]
    cms = [tt * reps[i] for i in range(_COLS)]
    ttis = [embp_ref[i] * wp + bp for i in range(_COLS)]
    blks = [
        lax.dot_general(
            ttis[i], cms[i], (((1,), (1,)), ((), ())),
            preferred_element_type=jnp.float32)
        for i in range(_COLS)
    ]
    for i in range(_COLS):
        t_ref[pl.ds(i * _UPAD, _UPAD), :] = blks[i]


_CHUNKS = (0, 3, 7, 13, 22)


def _sc_kernel(table_hbm, feat_hbm, sq_hbm, out_hbm, part_hbm,
               table_v, feat_v, sq_v, out_v, acc_v, accs_v, *sems):
    wid = lax.axis_index("s") * _NC + lax.axis_index("c")
    base = wid * _BPW

    cps = []
    for k in range(len(_CHUNKS) - 1):
        w0 = _CHUNKS[k] * _UPAD * _NCOLS
        w1 = _CHUNKS[k + 1] * _UPAD * _NCOLS
        cps.append(pltpu.async_copy(
            table_hbm.at[pl.ds(w0, w1 - w0)],
            table_v.at[pl.ds(w0, w1 - w0)], sems[k]))
    pltpu.sync_copy(feat_hbm.at[:, pl.ds(base, _BPW)], feat_v)
    pltpu.sync_copy(sq_hbm, sq_v)
    zeros = jnp.zeros((_LANES,), jnp.float32)
    for i in range(_COLS):
        acc_v[pl.ds(i * _LANES, _LANES)] = zeros

    for k in range(len(_CHUNKS) - 1):
        cps[k].wait()
        i0, i1 = _CHUNKS[k], _CHUNKS[k + 1]

        def group_body(g, carry, i0=i0, i1=i1, first=(k == 0)):
            g16 = g * _LANES
            fj2 = {j: feat_v[j, pl.ds(g16, _LANES)] + (_VOCAB * j)
                   for j in range(i0, _COLS)}
            accs = [jnp.zeros((_LANES,), jnp.float32)
                    for _ in range(_NACC)]
            n = 0
            if first:
                for i in range(_COLS):
                    fi = (feat_v[i, pl.ds(g16, _LANES)] + (_VOCAB * i)
                          if i not in fj2 else fj2[i])
                    sqg = plsc.load_gather(sq_v, [fi])
                    plsc.addupdate(acc_v.at[pl.ds(i * _LANES, _LANES)],
                                   sqg)
            for i in range(i0, i1):
                hi = fj2[i] * _NCOLS + ((_UPAD - _VOCAB) * _NCOLS * i)
                for j in range(i, _COLS):
                    idx = hi + fj2[j]
                    accs[n % _NACC] = accs[n % _NACC] + plsc.load_gather(
                        table_v, [idx])
                    n += 1
            tot = (accs[0] + accs[1]) + (accs[2] + accs[3])
            if first:
                out_v[pl.ds(g16, _LANES)] = tot
            else:
                plsc.addupdate(out_v.at[pl.ds(g16, _LANES)], tot)
            return carry

        lax.fori_loop(0, _GROUPS, group_body, 0)

    pltpu.sync_copy(out_v, out_hbm.at[pl.ds(base, _BPW)])
    lane = lax.iota(jnp.int32, _LANES)
    lo = jnp.zeros((_LANES,), jnp.float32)
    hi_v = jnp.zeros((_LANES,), jnp.float32)
    for i in range(_COLS):
        s_i = jnp.sum(acc_v[pl.ds(i * _LANES, _LANES)])
        if i < _LANES:
            lo = jnp.where(lane == i, s_i, lo)
        else:
            hi_v = jnp.where(lane == (i - _LANES), s_i, hi_v)
    accs_v[pl.ds(0, _LANES)] = lo
    accs_v[pl.ds(_LANES, _LANES)] = hi_v
    pltpu.sync_copy(accs_v, part_hbm.at[wid])


def _regs_kernel(p_ref, out_ref):
    x = p_ref[...]
    s = jnp.sum(x, axis=0, keepdims=True)
    lane = lax.broadcasted_iota(jnp.int32, (1, 32), 1)
    per_col = jnp.where(lane < _COLS, s, 0.0)
    out_ref[0, 0] = jnp.sum(jnp.sqrt(per_col)) * jnp.float32(
        2 * _COLS * _REG)


def kernel(features, emb, fc_w, w_p, b_p):
    embp = jnp.pad(emb, ((0, 0), (0, _UPAD - _VOCAB), (0, 0)))
    ep2 = embp.reshape(_NROWS, _D)
    wp2 = jnp.reshape(w_p, (1, 1))
    bp2 = jnp.reshape(b_p, (1, 1))

    table, sq = pl.pallas_call(
        _table_kernel,
        in_specs=[
            pl.BlockSpec(memory_space=pltpu.SMEM),
            pl.BlockSpec(memory_space=pltpu.SMEM),
            pl.BlockSpec(memory_space=pltpu.VMEM),
            pl.BlockSpec(memory_space=pltpu.VMEM),
            pl.BlockSpec(memory_space=pltpu.VMEM),
        ],
        out_shape=[
            jax.ShapeDtypeStruct((_NROWS, _NCOLS), jnp.float32),
            jax.ShapeDtypeStruct((_NCOLS, 1), jnp.float32),
        ],
    )(wp2, bp2, embp, ep2, fc_w)

    mesh = plsc.VectorSubcoreMesh(
        core_axis_name="c", subcore_axis_name="s",
        num_cores=_NC, num_subcores=_NS)
    sc_fn = pl.kernel(
        _sc_kernel,
        out_type=[
            jax.ShapeDtypeStruct((_B,), jnp.float32),
            jax.ShapeDtypeStruct((_NW, 32), jnp.float32),
        ],
        mesh=mesh,
        compiler_params=pltpu.CompilerParams(
            needs_layout_passes=False, skip_device_barrier=True),
        scratch_types=[
            pltpu.VMEM((_TABLE_WORDS,), jnp.float32),
            pltpu.VMEM((_COLS, _BPW), jnp.int32),
            pltpu.VMEM((_NCOLS,), jnp.float32),
            pltpu.VMEM((_BPW,), jnp.float32),
            pltpu.VMEM((_NROWS,), jnp.float32),
            pltpu.VMEM((32,), jnp.float32),
        ] + [pltpu.SemaphoreType.DMA] * (len(_CHUNKS) - 1),
    )
    out_flat, part = sc_fn(table.reshape(_TABLE_WORDS), features,
                           sq.reshape(_NCOLS))

    regs2 = pl.pallas_call(
        _regs_kernel,
        out_shape=jax.ShapeDtypeStruct((1, 1), jnp.float32),
        out_specs=pl.BlockSpec(memory_space=pltpu.SMEM),
    )(part)

    return out_flat.reshape(_B, 1), jnp.reshape(regs2, ())
